# baseline (speedup 1.0000x reference)
import functools
import math

import jax
import jax.numpy as jnp
from jax import lax
from jax.experimental import pallas as pl
from jax.experimental.pallas import tpu as pltpu

_TEMP = 2.0
_E = math.exp(1.0 / _TEMP)
_BIG = 1e9


def _dp_hits(os_, tg, mf, B, P, L):
    jidx = lax.broadcasted_iota(jnp.int32, (B, L), 1).astype(jnp.float32)
    row = jidx
    hits = []
    for i in range(P):
        if i > 0:
            sym = os_[:, i - 1 : i]
            neq = (sym != tg).astype(jnp.float32)
            e_rest = jnp.minimum(row[:, :-1] + neq[:, :-1], row[:, 1:] + 1.0)
            e0 = jnp.full((B, 1), float(i), dtype=jnp.float32)
            e = jnp.concatenate([e0, e_rest], axis=1)
            m = e - jidx
            s = 1
            while s < L:
                shifted = jnp.concatenate(
                    [jnp.full((B, s), _BIG, dtype=jnp.float32), m[:, :-s]], axis=1
                )
                m = jnp.minimum(m, shifted)
                s *= 2
            row = m + jidx
        dm = jnp.where(mf > 0.0, row, _BIG)
        mn = jnp.min(dm, axis=1, keepdims=True)
        hit = ((dm == mn) & (mf > 0.0)).astype(jnp.float32)
        hits.append(hit)
    return jnp.stack(hits, axis=1)


def _body(osym_ref, tgt_ref, maskf_ref, out_block_ref,
          o_ref, r_ref, *, B, P, L, V):
    b = pl.program_id(0)

    x = out_block_ref[0]
    trow = tgt_ref[pl.ds(b, 1), :]
    trowx = jnp.concatenate(
        [trow, jnp.full((1, 1), -1, jnp.int32)], axis=1
    )
    io = lax.broadcasted_iota(jnp.int32, (V, L + 1), 0)
    ci = lax.broadcasted_iota(jnp.int32, (V, L + 1), 1)
    w = ((io == trowx) | (ci == L)).astype(jnp.float32)
    r = jax.lax.dot(x, w, preferred_element_type=jnp.float32)
    r_ref[b] = r

    @pl.when(b == pl.num_programs(0) - 1)
    def _fin():
        os_ = osym_ref[...]
        tg = tgt_ref[...]
        mf = maskf_ref[...]
        hit = _dp_hits(os_, tg, mf, B, P, L)

        jl = lax.broadcasted_iota(jnp.int32, (B, L), 1)
        a = jnp.zeros((B, P, L), jnp.float32)
        for jp in range(L):
            same_row = ((tg[:, jp : jp + 1] == tg) & (jl > jp)).astype(
                jnp.float32
            )
            a = a + hit[:, :, jp : jp + 1] * same_row[:, None, :]
        contrib = hit * (a == 0.0).astype(jnp.float32)

        rr = r_ref[...]
        g = rr[:, :, :L]
        sall = rr[:, :, L]
        kk = jnp.sum(contrib, axis=-1)
        shit = jnp.sum(contrib * g, axis=-1)

        z = kk * _E + (float(V) - kk)
        kl = 0.5 * _E * kk / z - jnp.log(z) - ((_E - 1.0) * shit + sall) / z

        loss = kl * mf
        wsum = jnp.sum(mf, axis=-1)
        per_b = jnp.sum(loss, axis=-1) / (wsum + 1e-13)
        num = jnp.sum(per_b)
        cnt = jnp.sum((wsum > 0.0).astype(jnp.float32))
        o_ref[...] = jnp.full((1, 1), 0.0, jnp.float32) + num / (cnt + 1e-13)


def kernel(outputs, output_symbols, targets, mask):
    B, P, V = outputs.shape
    L = targets.shape[1]
    maskf = mask.astype(jnp.float32)

    body = functools.partial(_body, B=B, P=P, L=L, V=V)
    out = pl.pallas_call(
        body,
        grid=(B,),
        in_specs=[
            pl.BlockSpec((B, P), lambda b: (0, 0)),
            pl.BlockSpec((B, L), lambda b: (0, 0)),
            pl.BlockSpec((B, L), lambda b: (0, 0)),
            pl.BlockSpec((1, P, V), lambda b: (b, 0, 0)),
        ],
        out_specs=pl.BlockSpec((1, 1), lambda b: (0, 0)),
        out_shape=jax.ShapeDtypeStruct((1, 1), jnp.float32),
        scratch_shapes=[
            pltpu.VMEM((B, P, L + 1), jnp.float32),
        ],
    )(output_symbols, targets, maskf, outputs)
    return out[0, 0]


# speedup vs baseline: 1.2399x; 1.2399x over previous
import functools
import math

import jax
import jax.numpy as jnp
from jax import lax
from jax.experimental import pallas as pl
from jax.experimental.pallas import tpu as pltpu

_TEMP = 2.0
_E = math.exp(1.0 / _TEMP)
_BIG = 1e9


def _dp_hits(os_, tg, mf, B, P, L):
    jidx = lax.broadcasted_iota(jnp.int32, (B, L), 1).astype(jnp.float32)
    row = jidx
    hits = []
    for i in range(P):
        if i > 0:
            sym = os_[:, i - 1 : i]
            neq = (sym != tg).astype(jnp.float32)
            e_rest = jnp.minimum(row[:, :-1] + neq[:, :-1], row[:, 1:] + 1.0)
            e0 = jnp.full((B, 1), float(i), dtype=jnp.float32)
            e = jnp.concatenate([e0, e_rest], axis=1)
            m = e - jidx
            s = 1
            while s < L:
                shifted = jnp.concatenate(
                    [jnp.full((B, s), _BIG, dtype=jnp.float32), m[:, :-s]], axis=1
                )
                m = jnp.minimum(m, shifted)
                s *= 2
            row = m + jidx
        dm = jnp.where(mf > 0.0, row, _BIG)
        mn = jnp.min(dm, axis=1, keepdims=True)
        hit = ((dm == mn) & (mf > 0.0)).astype(jnp.float32)
        hits.append(hit)
    return jnp.stack(hits, axis=1)


def _body(osym_ref, tgt_ref, tcol_ref, maskf_ref, out_block_ref,
          o_ref, r_ref, *, B, P, L, V):
    b = pl.program_id(0)

    x = out_block_ref[0]
    tcol = tcol_ref[0]
    tcolx = jnp.concatenate(
        [tcol, jnp.full((1, 1), -1, jnp.int32)], axis=0
    )
    io = lax.broadcasted_iota(jnp.int32, (L + 1, V), 1)
    ri = lax.broadcasted_iota(jnp.int32, (L + 1, V), 0)
    w = ((io == tcolx) | (ri == L)).astype(jnp.float32)
    r = jax.lax.dot_general(
        x, w, (((1,), (1,)), ((), ())), preferred_element_type=jnp.float32
    )
    r_ref[b] = r

    @pl.when(b == pl.num_programs(0) - 1)
    def _fin():
        os_ = osym_ref[...]
        tg = tgt_ref[...]
        mf = maskf_ref[...]
        hit = _dp_hits(os_, tg, mf, B, P, L)

        jl = lax.broadcasted_iota(jnp.int32, (B, L), 1)
        a = jnp.zeros((B, P, L), jnp.float32)
        for jp in range(L):
            same_row = ((tg[:, jp : jp + 1] == tg) & (jl > jp)).astype(
                jnp.float32
            )
            a = a + hit[:, :, jp : jp + 1] * same_row[:, None, :]
        contrib = hit * (a == 0.0).astype(jnp.float32)

        rr = r_ref[...]
        g = rr[:, :, :L]
        sall = rr[:, :, L]
        kk = jnp.sum(contrib, axis=-1)
        shit = jnp.sum(contrib * g, axis=-1)

        z = kk * _E + (float(V) - kk)
        kl = 0.5 * _E * kk / z - jnp.log(z) - ((_E - 1.0) * shit + sall) / z

        loss = kl * mf
        wsum = jnp.sum(mf, axis=-1)
        per_b = jnp.sum(loss, axis=-1) / (wsum + 1e-13)
        num = jnp.sum(per_b)
        cnt = jnp.sum((wsum > 0.0).astype(jnp.float32))
        o_ref[...] = jnp.full((1, 1), 0.0, jnp.float32) + num / (cnt + 1e-13)


def kernel(outputs, output_symbols, targets, mask):
    B, P, V = outputs.shape
    L = targets.shape[1]
    maskf = mask.astype(jnp.float32)
    targets_col = targets[:, :, None]

    body = functools.partial(_body, B=B, P=P, L=L, V=V)
    out = pl.pallas_call(
        body,
        grid=(B,),
        in_specs=[
            pl.BlockSpec((B, P), lambda b: (0, 0)),
            pl.BlockSpec((B, L), lambda b: (0, 0)),
            pl.BlockSpec((1, L, 1), lambda b: (b, 0, 0)),
            pl.BlockSpec((B, L), lambda b: (0, 0)),
            pl.BlockSpec((1, P, V), lambda b: (b, 0, 0)),
        ],
        out_specs=pl.BlockSpec((1, 1), lambda b: (0, 0)),
        out_shape=jax.ShapeDtypeStruct((1, 1), jnp.float32),
        scratch_shapes=[
            pltpu.VMEM((B, P, L + 1), jnp.float32),
        ],
    )(output_symbols, targets, targets_col, maskf, outputs)
    return out[0, 0]


# speedup vs baseline: 1.2717x; 1.0256x over previous
import functools
import math

import jax
import jax.numpy as jnp
from jax import lax
from jax.experimental import pallas as pl
from jax.experimental.pallas import tpu as pltpu

_TEMP = 2.0
_E = math.exp(1.0 / _TEMP)
_BIG = 1e9


def _dp_hits(os_, tg, mf, B, P, L):
    jidx = lax.broadcasted_iota(jnp.int32, (B, L), 1).astype(jnp.float32)
    row = jidx
    hits = []
    for i in range(P):
        if i > 0:
            sym = os_[:, i - 1 : i]
            neq = (sym != tg).astype(jnp.float32)
            e_rest = jnp.minimum(row[:, :-1] + neq[:, :-1], row[:, 1:] + 1.0)
            e0 = jnp.full((B, 1), float(i), dtype=jnp.float32)
            e = jnp.concatenate([e0, e_rest], axis=1)
            m = e - jidx
            s = 1
            while s < L:
                shifted = jnp.concatenate(
                    [jnp.full((B, s), _BIG, dtype=jnp.float32), m[:, :-s]], axis=1
                )
                m = jnp.minimum(m, shifted)
                s *= 2
            row = m + jidx
        dm = jnp.where(mf > 0.0, row, _BIG)
        mn = jnp.min(dm, axis=1, keepdims=True)
        hit = ((dm == mn) & (mf > 0.0)).astype(jnp.float32)
        hits.append(hit)
    return jnp.stack(hits, axis=1)


def _body(osym_ref, tgt_ref, tcol_ref, maskf_ref, out_block_ref,
          o_ref, hit_ref, acc_ref, *, B, P, L, V):
    b = pl.program_id(0)

    @pl.when(b == 0)
    def _init():
        hit_ref[...] = _dp_hits(
            osym_ref[...], tgt_ref[...], maskf_ref[...], B, P, L
        )
        acc_ref[0] = 0.0
        acc_ref[1] = 0.0

    x = out_block_ref[0]
    tcol = tcol_ref[0]
    trow = tgt_ref[pl.ds(b, 1), :]
    tcolx = jnp.concatenate(
        [tcol, jnp.full((1, 1), -1, jnp.int32)], axis=0
    )
    io = lax.broadcasted_iota(jnp.int32, (L + 1, V), 1)
    ri = lax.broadcasted_iota(jnp.int32, (L + 1, V), 0)
    w = ((io == tcolx) | (ri == L)).astype(jnp.float32)
    r = jax.lax.dot_general(
        x, w, (((1,), (1,)), ((), ())), preferred_element_type=jnp.float32
    )
    g = r[:, :L]
    sall = r[:, L:]

    hit_b = hit_ref[b]
    ji = lax.broadcasted_iota(jnp.int32, (L, L), 0)
    jj = lax.broadcasted_iota(jnp.int32, (L, L), 1)
    m = ((tcol == trow) & (ji < jj)).astype(jnp.float32)
    a = jax.lax.dot(hit_b, m, preferred_element_type=jnp.float32)
    contrib = hit_b * (a == 0.0).astype(jnp.float32)

    kk = jnp.sum(contrib, axis=-1, keepdims=True)
    shit = jnp.sum(contrib * g, axis=-1, keepdims=True)
    z = kk * _E + (float(V) - kk)
    kl = 0.5 * _E * kk / z - jnp.log(z) - ((_E - 1.0) * shit + sall) / z

    mrow = maskf_ref[pl.ds(b, 1), :]
    wsum = jnp.sum(mrow)
    per_b = jnp.sum(kl[:, 0] * mrow[0]) / (wsum + 1e-13)
    acc_ref[0] += per_b
    acc_ref[1] += (wsum > 0.0).astype(jnp.float32)

    @pl.when(b == pl.num_programs(0) - 1)
    def _fin():
        val = acc_ref[0] / (acc_ref[1] + 1e-13)
        o_ref[...] = jnp.full((1, 1), 0.0, jnp.float32) + val


def kernel(outputs, output_symbols, targets, mask):
    B, P, V = outputs.shape
    L = targets.shape[1]
    maskf = mask.astype(jnp.float32)
    targets_col = targets[:, :, None]

    body = functools.partial(_body, B=B, P=P, L=L, V=V)
    out = pl.pallas_call(
        body,
        grid=(B,),
        in_specs=[
            pl.BlockSpec((B, P), lambda b: (0, 0)),
            pl.BlockSpec((B, L), lambda b: (0, 0)),
            pl.BlockSpec((1, L, 1), lambda b: (b, 0, 0)),
            pl.BlockSpec((B, L), lambda b: (0, 0)),
            pl.BlockSpec((1, P, V), lambda b: (b, 0, 0)),
        ],
        out_specs=pl.BlockSpec((1, 1), lambda b: (0, 0)),
        out_shape=jax.ShapeDtypeStruct((1, 1), jnp.float32),
        scratch_shapes=[
            pltpu.VMEM((B, P, L), jnp.float32),
            pltpu.SMEM((2,), jnp.float32),
        ],
    )(output_symbols, targets, targets_col, maskf, outputs)
    return out[0, 0]


# speedup vs baseline: 2.2495x; 1.7689x over previous
import functools
import math

import jax
import jax.numpy as jnp
from jax import lax
from jax.experimental import pallas as pl
from jax.experimental.pallas import tpu as pltpu

_TEMP = 2.0
_E = math.exp(1.0 / _TEMP)
_BIG = 1e9


def _dp_hits(os_, tg, mf, B, P, L):
    jidx = lax.broadcasted_iota(jnp.int32, (B, L), 1).astype(jnp.float32)
    row = jidx
    hits = []
    for i in range(P):
        if i > 0:
            sym = os_[:, i - 1 : i]
            neq = (sym != tg).astype(jnp.float32)
            e_rest = jnp.minimum(row[:, :-1] + neq[:, :-1], row[:, 1:] + 1.0)
            e0 = jnp.full((B, 1), float(i), dtype=jnp.float32)
            e = jnp.concatenate([e0, e_rest], axis=1)
            m = e - jidx
            s = 1
            while s < L:
                shifted = jnp.concatenate(
                    [jnp.full((B, s), _BIG, dtype=jnp.float32), m[:, :-s]], axis=1
                )
                m = jnp.minimum(m, shifted)
                s *= 2
            row = m + jidx
        dm = jnp.where(mf > 0.0, row, _BIG)
        mn = jnp.min(dm, axis=1, keepdims=True)
        hit = ((dm == mn) & (mf > 0.0)).astype(jnp.float32)
        hits.append(hit)
    return jnp.stack(hits, axis=1)


def _body(osym_ref, tgt_ref, tcol_ref, maskf_ref, out_block_ref,
          o_ref, hit_ref, acc_ref, *, B, P, L, V, BB):
    i = pl.program_id(0)

    @pl.when(i == 0)
    def _init():
        hit_ref[...] = _dp_hits(
            osym_ref[...], tgt_ref[...], maskf_ref[...], B, P, L
        )
        acc_ref[0] = 0.0
        acc_ref[1] = 0.0

    io = lax.broadcasted_iota(jnp.int32, (L + 1, V), 1)
    ri = lax.broadcasted_iota(jnp.int32, (L + 1, V), 0)
    ji = lax.broadcasted_iota(jnp.int32, (L, L), 0)
    jj = lax.broadcasted_iota(jnp.int32, (L, L), 1)

    for bb in range(BB):
        b = i * BB + bb
        x = out_block_ref[bb]
        tcol = tcol_ref[bb]
        trow = tgt_ref[pl.ds(b, 1), :]
        tcolx = jnp.concatenate(
            [tcol, jnp.full((1, 1), -1, jnp.int32)], axis=0
        )
        w = ((io == tcolx) | (ri == L)).astype(jnp.float32)
        r = jax.lax.dot_general(
            x, w, (((1,), (1,)), ((), ())), preferred_element_type=jnp.float32
        )
        g = r[:, :L]
        sall = r[:, L:]

        hit_b = hit_ref[b]
        m = ((tcol == trow) & (ji < jj)).astype(jnp.float32)
        a = jax.lax.dot(hit_b, m, preferred_element_type=jnp.float32)
        contrib = hit_b * (a == 0.0).astype(jnp.float32)

        kk = jnp.sum(contrib, axis=-1, keepdims=True)
        shit = jnp.sum(contrib * g, axis=-1, keepdims=True)
        z = kk * _E + (float(V) - kk)
        kl = 0.5 * _E * kk / z - jnp.log(z) - ((_E - 1.0) * shit + sall) / z

        mrow = maskf_ref[pl.ds(b, 1), :]
        wsum = jnp.sum(mrow)
        per_b = jnp.sum(kl[:, 0] * mrow[0]) / (wsum + 1e-13)
        acc_ref[0] += per_b
        acc_ref[1] += (wsum > 0.0).astype(jnp.float32)

    @pl.when(i == pl.num_programs(0) - 1)
    def _fin():
        val = acc_ref[0] / (acc_ref[1] + 1e-13)
        o_ref[...] = jnp.full((1, 1), 0.0, jnp.float32) + val


def kernel(outputs, output_symbols, targets, mask):
    B, P, V = outputs.shape
    L = targets.shape[1]
    maskf = mask.astype(jnp.float32)
    targets_col = targets[:, :, None]

    BB = 8
    body = functools.partial(_body, B=B, P=P, L=L, V=V, BB=BB)
    out = pl.pallas_call(
        body,
        grid=(B // BB,),
        in_specs=[
            pl.BlockSpec((B, P), lambda b: (0, 0)),
            pl.BlockSpec((B, L), lambda b: (0, 0)),
            pl.BlockSpec((BB, L, 1), lambda b: (b, 0, 0)),
            pl.BlockSpec((B, L), lambda b: (0, 0)),
            pl.BlockSpec((BB, P, V), lambda b: (b, 0, 0)),
        ],
        out_specs=pl.BlockSpec((1, 1), lambda b: (0, 0)),
        out_shape=jax.ShapeDtypeStruct((1, 1), jnp.float32),
        scratch_shapes=[
            pltpu.VMEM((B, P, L), jnp.float32),
            pltpu.SMEM((2,), jnp.float32),
        ],
    )(output_symbols, targets, targets_col, maskf, outputs)
    return out[0, 0]
